# baseline (speedup 1.0000x reference)
import jax
import jax.numpy as jnp
from jax.experimental import pallas as pl
from jax.experimental.pallas import tpu as pltpu


def _fused_kernel(adj_ref, feature_ref, w_ref, b_ref, out_ref, support_ref):
    i = pl.program_id(0)

    @pl.when(i == 0)
    def _():
        support_ref[...] = jnp.maximum(
            jnp.dot(feature_ref[...], w_ref[...], preferred_element_type=jnp.float32),
            0.0,
        )

    @pl.when(i > 0)
    def _():
        br = adj_ref.shape[0]
        j = i - 1
        acc = jnp.dot(
            adj_ref[...], support_ref[...], preferred_element_type=jnp.float32
        )
        out_ref[...] = acc + support_ref[pl.ds(j * br, br), :] + b_ref[...]


def kernel(feature, adj, W, b):
    n, f_in = feature.shape
    d = W.shape[1]
    b2 = b.reshape(1, d)

    br = 400
    grid = (n // br + 1,)

    def _blk(i):
        return (jnp.maximum(i - 1, 0), 0)

    out = pl.pallas_call(
        _fused_kernel,
        grid=grid,
        in_specs=[
            pl.BlockSpec((br, n), _blk),
            pl.BlockSpec((n, f_in), lambda i: (0, 0)),
            pl.BlockSpec((f_in, d), lambda i: (0, 0)),
            pl.BlockSpec((1, d), lambda i: (0, 0)),
        ],
        out_specs=pl.BlockSpec((br, d), _blk),
        out_shape=jax.ShapeDtypeStruct((n, d), jnp.float32),
        scratch_shapes=[
            pltpu.VMEM((n, d), jnp.float32),
        ],
        compiler_params=pltpu.CompilerParams(
            dimension_semantics=("arbitrary",),
            skip_device_barrier=True,
        ),
    )(adj, feature, W, b2)

    return out


# speedup vs baseline: 1.0219x; 1.0219x over previous
import jax
import jax.numpy as jnp
from jax.experimental import pallas as pl
from jax.experimental.pallas import tpu as pltpu


def _fused_kernel(adj_ref, feature_ref, w_ref, b_ref, out_ref, support_ref):
    i = pl.program_id(0)

    @pl.when(i == 0)
    def _():
        support_ref[...] = jnp.maximum(
            jnp.dot(feature_ref[...], w_ref[...], preferred_element_type=jnp.float32),
            0.0,
        )

    br = out_ref.shape[0]
    acc = jnp.dot(adj_ref[...], support_ref[...], preferred_element_type=jnp.float32)
    out_ref[...] = acc + support_ref[pl.ds(i * br, br), :] + b_ref[...]


def kernel(feature, adj, W, b):
    n, f_in = feature.shape
    d = W.shape[1]
    b2 = b.reshape(1, d)

    br = 400
    grid = (n // br,)

    out = pl.pallas_call(
        _fused_kernel,
        grid=grid,
        in_specs=[
            pl.BlockSpec((br, n), lambda i: (i, 0)),
            pl.BlockSpec((n, f_in), lambda i: (0, 0)),
            pl.BlockSpec((f_in, d), lambda i: (0, 0)),
            pl.BlockSpec((1, d), lambda i: (0, 0)),
        ],
        out_specs=pl.BlockSpec((br, d), lambda i: (i, 0)),
        out_shape=jax.ShapeDtypeStruct((n, d), jnp.float32),
        scratch_shapes=[
            pltpu.VMEM((n, d), jnp.float32),
        ],
        compiler_params=pltpu.CompilerParams(
            dimension_semantics=("arbitrary",),
            skip_device_barrier=True,
        ),
    )(adj, feature, W, b2)

    return out
